# Initial kernel scaffold; baseline (speedup 1.0000x reference)
import jax
import jax.numpy as jnp
from jax.experimental import pallas as pl


def kernel(edge_index, edge_values, emb_user, emb_item):
    raise NotImplementedError("write your pallas kernel here")


# speedup vs baseline: 5.1258x; 5.1258x over previous
import jax
import jax.numpy as jnp
from jax.experimental import pallas as pl
from jax.experimental.pallas import tpu as pltpu
from jax.experimental.pallas import tpu_sc as plsc

N_USERS = 50000
N = 100000
NP = 102400
D = 32
H = 16
NS = 16
E = 1600000
EPAD = E + 4096
C = 704
ROWS_T = NP // NS

_f32 = jnp.float32
_i32 = jnp.int32


def _sc_propagate(src, rows_s, cols_s, vals_s, starts2):
    def body(src_ref, rows_h, cols_h, vals_h, st2_h, e1, e2, e3,
             accf, colb, rowb, valb, gath, sbuf, sem_in, sem_g):
        c = jax.lax.axis_index("c")
        s = jax.lax.axis_index("s")
        base = s * ROWS_T
        iota = jax.lax.iota(_i32, 16)

        pltpu.sync_copy(st2_h, sbuf)
        s0 = sbuf[0, :]
        s1 = sbuf[1, :]
        st = jnp.int32(0)
        en = jnp.int32(0)
        for j in range(NS):
            st = jnp.where(s == j, s0[j], st)
            en = jnp.where(s == j, s1[j], en)
        st8 = jax.lax.bitwise_and(st, jnp.int32(~7))
        nchunks = (en - st8 + (C - 1)) // C
        npairs = (nchunks + 1) // 2

        def issue_in(i, b):
            off = pl.multiple_of(st8 + i * C, 8)
            pltpu.async_copy(cols_h.at[pl.ds(off, C)], colb.at[b],
                             sem_in.at[b])
            pltpu.async_copy(vals_h.at[pl.ds(off, C)], valb.at[b],
                             sem_in.at[b])
            pltpu.async_copy(rows_h.at[pl.ds(off, C)], rowb.at[b],
                             sem_in.at[b])

        def wait_in(i, b):
            off = pl.multiple_of(st8 + i * C, 8)
            pltpu.make_async_copy(cols_h.at[pl.ds(off, C)], colb.at[b],
                                  sem_in.at[b]).wait()
            pltpu.make_async_copy(vals_h.at[pl.ds(off, C)], valb.at[b],
                                  sem_in.at[b]).wait()
            pltpu.make_async_copy(rows_h.at[pl.ds(off, C)], rowb.at[b],
                                  sem_in.at[b]).wait()

        def edge_pass(layer_src):
            def issue_gather(b):
                pltpu.async_copy(layer_src.at[c].at[colb.at[b]],
                                 gath.at[b], sem_g.at[b])

            def wait_gather(b):
                pltpu.make_async_copy(layer_src.at[c].at[colb.at[b]],
                                      gath.at[b], sem_g.at[b]).wait()

            def compute(i, b):
                chunk0 = st8 + i * C

                @pl.loop(0, C // 16)
                def _(g):
                    e0i = g * 16
                    rowv = rowb[b, pl.ds(e0i, 16)]
                    rowv = jnp.clip(rowv - base, 0, ROWS_T - 1)
                    vv = valb[b, pl.ds(e0i, 16)]
                    gidx = chunk0 + e0i + iota
                    ok = jnp.logical_and(gidx >= st, gidx < en)
                    vv = jnp.where(ok, vv, jnp.zeros((16,), _f32))
                    for j in range(16):
                        msg = gath[b, e0i + j, :] * vv[j]
                        plsc.addupdate_scatter(
                            accf, [jnp.full((16,), rowv[j], _i32), iota], msg)

            issue_in(0, 0)
            wait_in(0, 0)
            issue_gather(0)

            @pl.loop(0, npairs)
            def _(k):
                i = 2 * k
                issue_in(i + 1, 1)
                wait_gather(0)
                wait_in(i + 1, 1)
                issue_gather(1)
                compute(i, 0)
                issue_in(i + 2, 0)
                wait_gather(1)
                wait_in(i + 2, 0)
                issue_gather(0)
                compute(i + 1, 1)

            wait_gather(0)

        for layer_src, layer_dst in ((src_ref, e1), (e1, e2), (e2, e3)):
            @pl.loop(0, ROWS_T)
            def _(r):
                accf[r, :] = jnp.zeros((H,), _f32)

            edge_pass(layer_src)
            pltpu.sync_copy(accf, layer_dst.at[c, pl.ds(base, ROWS_T)])
            plsc.subcore_barrier()

    out_t = jax.ShapeDtypeStruct((2, NP, H), _f32)
    mesh = plsc.VectorSubcoreMesh(core_axis_name="c", subcore_axis_name="s")
    f = pl.kernel(
        body,
        out_type=(out_t, out_t, out_t),
        mesh=mesh,
        compiler_params=pltpu.CompilerParams(use_tc_tiling_on_sc=False,
                                             needs_layout_passes=False),
        scratch_types=[
            pltpu.VMEM((ROWS_T, H), _f32),
            pltpu.VMEM((2, C), _i32),
            pltpu.VMEM((2, C), _i32),
            pltpu.VMEM((2, C), _f32),
            pltpu.VMEM((2, C, H), _f32),
            pltpu.VMEM((2, 16), _i32),
            pltpu.SemaphoreType.DMA((2,)),
            pltpu.SemaphoreType.DMA((2,)),
        ],
    )
    return f(src, rows_s, cols_s, vals_s, starts2)


def _tc_mean(e0, e1, e2, e3):
    def body(a, b, c, d, o):
        o[...] = 0.25 * (a[...] + b[...] + c[...] + d[...])

    blk = pl.BlockSpec((1024, 128), lambda i: (i, 0))
    return pl.pallas_call(
        body,
        grid=(25,),
        in_specs=[blk, blk, blk, blk],
        out_specs=blk,
        out_shape=jax.ShapeDtypeStruct((25600, 128), _f32),
    )(e0, e1, e2, e3)


def kernel(edge_index, edge_values, emb_user, emb_item):
    rows, cols, vals = edge_index[0], edge_index[1], edge_values
    rows_s, cols_s, vals_s = jax.lax.sort([rows, cols, vals], num_keys=1)
    bounds = jnp.arange(0, NP + 1, ROWS_T, dtype=_i32)
    starts = jnp.searchsorted(rows_s, bounds).astype(_i32)
    starts2 = jnp.stack([starts[:NS], starts[1:NS + 1]])
    pad = EPAD - E
    rows_s = jnp.concatenate([rows_s, jnp.zeros((pad,), _i32)])
    cols_s = jnp.concatenate([cols_s, jnp.zeros((pad,), _i32)])
    vals_s = jnp.concatenate([vals_s, jnp.zeros((pad,), _f32)])

    all_emb = jnp.concatenate([emb_user, emb_item], axis=0)
    src = all_emb.reshape(N, 2, H).transpose(1, 0, 2)
    src = jnp.pad(src, ((0, 0), (0, NP - N), (0, 0)))
    e1, e2, e3 = _sc_propagate(src, rows_s, cols_s, vals_s, starts2)
    flat = (2 * NP * H) // 128
    mean = _tc_mean(src.reshape(flat, 128), e1.reshape(flat, 128),
                    e2.reshape(flat, 128), e3.reshape(flat, 128))
    out = mean.reshape(2, NP, H)[:, :N].transpose(1, 0, 2).reshape(N, D)
    return out[:N_USERS], out[N_USERS:]


# speedup vs baseline: 5.6877x; 1.1096x over previous
import jax
import jax.numpy as jnp
from jax.experimental import pallas as pl
from jax.experimental.pallas import tpu as pltpu
from jax.experimental.pallas import tpu_sc as plsc

N_USERS = 50000
N = 100000
NP = 102400
D = 32
H = 16
NS = 16
E = 1600000
EPAD = E + 4096
C = 704
ROWS_T = NP // NS
S = 16
L = E // S

_f32 = jnp.float32
_i32 = jnp.int32


def _sc_propagate(src, rows_s, cols_s, vals_s, starts2):
    def body(src_ref, rows_h, cols_h, vals_h, st2_h, e1, e2, e3,
             accf, colb, rowb, valb, gath, sbuf, sem_in, sem_g):
        c = jax.lax.axis_index("c")
        s = jax.lax.axis_index("s")
        base = s * ROWS_T
        iota = jax.lax.iota(_i32, 16)

        pltpu.sync_copy(st2_h.at[s], sbuf)
        s0 = sbuf[0, :]
        s1 = sbuf[1, :]

        def edge_pass(layer_src):
            @pl.loop(0, S)
            def _(slc):
                st = jnp.int32(0)
                en = jnp.int32(0)
                for j in range(S):
                    st = jnp.where(slc == j, s0[j], st)
                    en = jnp.where(slc == j, s1[j], en)
                st8 = jax.lax.bitwise_and(st, jnp.int32(~7))
                nchunks = (en - st8 + (C - 1)) // C
                npairs = (nchunks + 1) // 2

                def issue_in(i, b):
                    off = pl.multiple_of(st8 + i * C, 8)
                    pltpu.async_copy(cols_h.at[pl.ds(off, C)], colb.at[b],
                                     sem_in.at[b])
                    pltpu.async_copy(vals_h.at[pl.ds(off, C)], valb.at[b],
                                     sem_in.at[b])
                    pltpu.async_copy(rows_h.at[pl.ds(off, C)], rowb.at[b],
                                     sem_in.at[b])

                def wait_in(i, b):
                    off = pl.multiple_of(st8 + i * C, 8)
                    pltpu.make_async_copy(cols_h.at[pl.ds(off, C)],
                                          colb.at[b], sem_in.at[b]).wait()
                    pltpu.make_async_copy(vals_h.at[pl.ds(off, C)],
                                          valb.at[b], sem_in.at[b]).wait()
                    pltpu.make_async_copy(rows_h.at[pl.ds(off, C)],
                                          rowb.at[b], sem_in.at[b]).wait()

                def issue_gather(b):
                    pltpu.async_copy(layer_src.at[c].at[colb.at[b]],
                                     gath.at[b], sem_g.at[b])

                def wait_gather(b):
                    pltpu.make_async_copy(layer_src.at[c].at[colb.at[b]],
                                          gath.at[b], sem_g.at[b]).wait()

                def compute(i, b):
                    chunk0 = st8 + i * C

                    @pl.loop(0, C // 16)
                    def _(g):
                        e0i = g * 16
                        rowv = rowb[b, pl.ds(e0i, 16)]
                        rowv = jnp.clip(rowv - base, 0, ROWS_T - 1)
                        vv = valb[b, pl.ds(e0i, 16)]
                        gidx = chunk0 + e0i + iota
                        ok = jnp.logical_and(gidx >= st, gidx < en)
                        vv = jnp.where(ok, vv, jnp.zeros((16,), _f32))
                        for j in range(16):
                            msg = gath[b, e0i + j, :] * vv[j]
                            plsc.addupdate_scatter(
                                accf,
                                [jnp.full((16,), rowv[j], _i32), iota], msg)

                issue_in(0, 0)
                wait_in(0, 0)
                issue_gather(0)

                @pl.loop(0, npairs)
                def _(k):
                    i = 2 * k
                    issue_in(i + 1, 1)
                    wait_gather(0)
                    wait_in(i + 1, 1)
                    issue_gather(1)
                    compute(i, 0)
                    issue_in(i + 2, 0)
                    wait_gather(1)
                    wait_in(i + 2, 0)
                    issue_gather(0)
                    compute(i + 1, 1)

                wait_gather(0)

        for layer_src, layer_dst in ((src_ref, e1), (e1, e2), (e2, e3)):
            @pl.loop(0, ROWS_T)
            def _(r):
                accf[r, :] = jnp.zeros((H,), _f32)

            edge_pass(layer_src)
            pltpu.sync_copy(accf, layer_dst.at[c, pl.ds(base, ROWS_T)])
            plsc.subcore_barrier()

    out_t = jax.ShapeDtypeStruct((2, NP, H), _f32)
    mesh = plsc.VectorSubcoreMesh(core_axis_name="c", subcore_axis_name="s")
    f = pl.kernel(
        body,
        out_type=(out_t, out_t, out_t),
        mesh=mesh,
        compiler_params=pltpu.CompilerParams(use_tc_tiling_on_sc=False,
                                             needs_layout_passes=False),
        scratch_types=[
            pltpu.VMEM((ROWS_T, H), _f32),
            pltpu.VMEM((2, C), _i32),
            pltpu.VMEM((2, C), _i32),
            pltpu.VMEM((2, C), _f32),
            pltpu.VMEM((2, C, H), _f32),
            pltpu.VMEM((2, 16), _i32),
            pltpu.SemaphoreType.DMA((2,)),
            pltpu.SemaphoreType.DMA((2,)),
        ],
    )
    return f(src, rows_s, cols_s, vals_s, starts2)


def _tc_mean(e0, e1, e2, e3):
    def body(a, b, c, d, o):
        o[...] = 0.25 * (a[...] + b[...] + c[...] + d[...])

    blk = pl.BlockSpec((1024, 128), lambda i: (i, 0))
    return pl.pallas_call(
        body,
        grid=(25,),
        in_specs=[blk, blk, blk, blk],
        out_specs=blk,
        out_shape=jax.ShapeDtypeStruct((25600, 128), _f32),
    )(e0, e1, e2, e3)


def kernel(edge_index, edge_values, emb_user, emb_item):
    rows, cols, vals = edge_index[0], edge_index[1], edge_values
    r3, c3, v3 = (rows.reshape(S, L), cols.reshape(S, L),
                  vals.reshape(S, L))
    rs, cs, vs = jax.lax.sort([r3, c3, v3], num_keys=1)
    bounds = jnp.arange(0, NP + 1, ROWS_T, dtype=_i32)
    starts = jax.vmap(lambda r: jnp.searchsorted(r, bounds))(rs)
    glob = starts.astype(_i32) + (jnp.arange(S, dtype=_i32) * L)[:, None]
    starts2 = jnp.stack([glob[:, :NS].T, glob[:, 1:NS + 1].T], axis=1)
    pad = EPAD - E
    rows_s = jnp.concatenate([rs.reshape(E), jnp.zeros((pad,), _i32)])
    cols_s = jnp.concatenate([cs.reshape(E), jnp.zeros((pad,), _i32)])
    vals_s = jnp.concatenate([vs.reshape(E), jnp.zeros((pad,), _f32)])

    all_emb = jnp.concatenate([emb_user, emb_item], axis=0)
    src = all_emb.reshape(N, 2, H).transpose(1, 0, 2)
    src = jnp.pad(src, ((0, 0), (0, NP - N), (0, 0)))
    e1, e2, e3 = _sc_propagate(src, rows_s, cols_s, vals_s, starts2)
    flat = (2 * NP * H) // 128
    mean = _tc_mean(src.reshape(flat, 128), e1.reshape(flat, 128),
                    e2.reshape(flat, 128), e3.reshape(flat, 128))
    out = mean.reshape(2, NP, H)[:, :N].transpose(1, 0, 2).reshape(N, D)
    return out[:N_USERS], out[N_USERS:]
